# baseline (speedup 1.0000x reference)
import functools

import jax
import jax.numpy as jnp
from jax import lax
from jax.experimental import pallas as pl
from jax.experimental.pallas import tpu as pltpu
from jax.experimental.pallas import tpu_sc as plsc

NUM_EMBEDDINGS = 100000
NB = 4096
SL = 50
D = 128
B = NB * SL
NC = 2
NS = 16
NW = NC * NS
B_PER_W = B // NW
IPR = 64
NGATH = B_PER_W // IPR
G = 6
NBUF = 2
GROUPS = [G] * (NGATH // G) + ([NGATH % G] if NGATH % G else [])
S = len(GROUPS)


def _emb_body(idx_hbm, table_hbm, out_hbm, idx_v, *scratch):
    bufs = scratch[:NBUF]
    gsems = scratch[NBUF:2 * NBUF]
    osems = scratch[2 * NBUF:3 * NBUF]
    wid = lax.axis_index("s") * NC + lax.axis_index("c")
    base = wid * B_PER_W
    pltpu.sync_copy(idx_hbm.at[pl.ds(base, B_PER_W)], idx_v)

    starts = [sum(GROUPS[:g]) for g in range(S)]

    def fire_out(g):
        n = GROUPS[g] * IPR
        return pltpu.async_copy(
            bufs[g % NBUF].at[pl.ds(0, n)],
            out_hbm.at[pl.ds(base + starts[g] * IPR, n)],
            osems[g % NBUF],
        )

    gath = {}
    outc = {}
    for g in range(S):
        b = g % NBUF
        if g >= NBUF:
            outc[g - NBUF].wait()
        gath[g] = [
            pltpu.async_copy(
                table_hbm.at[idx_v.at[pl.ds((starts[g] + j) * IPR, IPR)]],
                bufs[b].at[pl.ds(j * IPR, IPR)],
                gsems[b],
            )
            for j in range(GROUPS[g])
        ]
        if g >= 1:
            for c in gath[g - 1]:
                c.wait()
            outc[g - 1] = fire_out(g - 1)
    for c in gath[S - 1]:
        c.wait()
    outc[S - 1] = fire_out(S - 1)
    for g in range(max(0, S - NBUF), S):
        outc[g].wait()


@jax.jit
def _embedding_lookup(idx, weight):
    mesh = plsc.VectorSubcoreMesh(core_axis_name="c", subcore_axis_name="s")
    fn = functools.partial(
        pl.kernel,
        mesh=mesh,
        out_type=jax.ShapeDtypeStruct((B, D), jnp.float32),
        scratch_types=(
            [pltpu.VMEM((B_PER_W,), jnp.int32)]
            + [pltpu.VMEM((G * IPR, D), jnp.float32)] * NBUF
            + [pltpu.SemaphoreType.DMA] * (2 * NBUF)
        ),
    )(_emb_body)
    return fn(idx, weight)


def kernel(input_, weight):
    idx = input_.astype(jnp.int32).T.reshape(B)
    out = _embedding_lookup(idx, weight)
    return out.reshape(SL, NB, D).transpose(1, 0, 2)


# speedup vs baseline: 1.0166x; 1.0166x over previous
import functools

import jax
import jax.numpy as jnp
from jax import lax
from jax.experimental import pallas as pl
from jax.experimental.pallas import tpu as pltpu
from jax.experimental.pallas import tpu_sc as plsc

NUM_EMBEDDINGS = 100000
NB = 4096
SL = 50
D = 128
B = NB * SL
NC = 2
NS = 16
NW = NC * NS
B_PER_W = B // NW
IPR = 128
NGATH = B_PER_W // IPR
G = 3
NBUF = 2
GROUPS = [G] * (NGATH // G) + ([NGATH % G] if NGATH % G else [])
S = len(GROUPS)


def _emb_body(idx_hbm, table_hbm, out_hbm, idx_v, *scratch):
    bufs = scratch[:NBUF]
    gsems = scratch[NBUF:2 * NBUF]
    osems = scratch[2 * NBUF:3 * NBUF]
    wid = lax.axis_index("s") * NC + lax.axis_index("c")
    base = wid * B_PER_W
    pltpu.sync_copy(idx_hbm.at[pl.ds(base, B_PER_W)], idx_v)

    starts = [sum(GROUPS[:g]) for g in range(S)]

    def fire_out(g):
        n = GROUPS[g] * IPR
        return pltpu.async_copy(
            bufs[g % NBUF].at[pl.ds(0, n)],
            out_hbm.at[pl.ds(base + starts[g] * IPR, n)],
            osems[g % NBUF],
        )

    gath = {}
    outc = {}
    for g in range(S):
        b = g % NBUF
        if g >= NBUF:
            outc[g - NBUF].wait()
        gath[g] = [
            pltpu.async_copy(
                table_hbm.at[idx_v.at[pl.ds((starts[g] + j) * IPR, IPR)]],
                bufs[b].at[pl.ds(j * IPR, IPR)],
                gsems[b],
            )
            for j in range(GROUPS[g])
        ]
        if g >= 1:
            for c in gath[g - 1]:
                c.wait()
            outc[g - 1] = fire_out(g - 1)
    for c in gath[S - 1]:
        c.wait()
    outc[S - 1] = fire_out(S - 1)
    for g in range(max(0, S - NBUF), S):
        outc[g].wait()


@jax.jit
def _embedding_lookup(idx, weight):
    mesh = plsc.VectorSubcoreMesh(core_axis_name="c", subcore_axis_name="s")
    fn = functools.partial(
        pl.kernel,
        mesh=mesh,
        out_type=jax.ShapeDtypeStruct((B, D), jnp.float32),
        scratch_types=(
            [pltpu.VMEM((B_PER_W,), jnp.int32)]
            + [pltpu.VMEM((G * IPR, D), jnp.float32)] * NBUF
            + [pltpu.SemaphoreType.DMA] * (2 * NBUF)
        ),
    )(_emb_body)
    return fn(idx, weight)


def kernel(input_, weight):
    idx = input_.astype(jnp.int32).T.reshape(B)
    out = _embedding_lookup(idx, weight)
    return out.reshape(SL, NB, D).transpose(1, 0, 2)
